# baseline (speedup 1.0000x reference)
import functools

import jax
import jax.numpy as jnp
from jax import lax
from jax.experimental import pallas as pl
from jax.experimental.pallas import tpu as pltpu
from jax.experimental.pallas import tpu_sc as plsc

BATCH = 1024
IN = 100000
OUT = 1024

NW = 32
RPW = BATCH // NW
PERIODS_PER_CHUNK = 8
CH = PERIODS_PER_CHUNK * OUT
NCH = 12
TAIL = IN - NCH * CH
NBUF = 4

_mesh = plsc.VectorSubcoreMesh(core_axis_name="c", subcore_axis_name="s")


@functools.partial(
    pl.kernel,
    mesh=_mesh,
    out_type=jax.ShapeDtypeStruct((BATCH * OUT,), jnp.float32),
    scratch_types=[
        pltpu.VMEM((CH,), jnp.float32),
        pltpu.VMEM((CH,), jnp.float32),
        pltpu.VMEM((CH,), jnp.float32),
        pltpu.VMEM((CH,), jnp.float32),
        pltpu.VMEM((TAIL,), jnp.float32),
        pltpu.VMEM((OUT,), jnp.float32),
        pltpu.SemaphoreType.DMA,
        pltpu.SemaphoreType.DMA,
        pltpu.SemaphoreType.DMA,
        pltpu.SemaphoreType.DMA,
        pltpu.SemaphoreType.DMA,
    ],
)
def _sc_kernel(x_hbm, out_hbm, b0, b1, b2, b3, tbuf, acc, s0, s1, s2, s3, st):
    bufs = (b0, b1, b2, b3)
    sems = (s0, s1, s2, s3)
    wid = lax.axis_index("s") * 2 + lax.axis_index("c")

    def start_chunk(row, c, b):
        off = pl.multiple_of(row * IN + c * CH, 8)
        pltpu.async_copy(x_hbm.at[pl.ds(off, CH)], bufs[b], sems[b])

    def wait_chunk(b):
        pltpu.make_async_copy(x_hbm.at[pl.ds(0, CH)], bufs[b], sems[b]).wait()

    def accum_chunk(b):
        for j in range(OUT // 16):
            v = bufs[b][pl.ds(j * 16, 16)]
            for p in range(1, PERIODS_PER_CHUNK):
                v = v + bufs[b][pl.ds(p * OUT + j * 16, 16)]
            plsc.addupdate(acc.at[pl.ds(j * 16, 16)], v)

    zero = jnp.zeros((16,), jnp.float32)

    def row_body(r, carry):
        row = wid * RPW + r
        toff = pl.multiple_of(row * IN + NCH * CH, 8)
        pltpu.async_copy(x_hbm.at[pl.ds(toff, TAIL)], tbuf, st)
        for b in range(NBUF):
            start_chunk(row, b, b)
        for j in range(OUT // 16):
            acc[pl.ds(j * 16, 16)] = zero

        def chunk_body(i, carry2):
            for b in range(NBUF):
                c = i * NBUF + b
                wait_chunk(b)
                accum_chunk(b)

                @pl.when(c + NBUF < NCH)
                def _():
                    start_chunk(row, c + NBUF, b)

            return carry2

        lax.fori_loop(0, NCH // NBUF, chunk_body, 0, unroll=False)

        pltpu.make_async_copy(x_hbm.at[pl.ds(0, TAIL)], tbuf, st).wait()
        for j in range(OUT // 16):
            v = tbuf[pl.ds(j * 16, 16)]
            plsc.addupdate(acc.at[pl.ds(j * 16, 16)], v)
        for j in range((TAIL - OUT) // 16):
            v = tbuf[pl.ds(OUT + j * 16, 16)]
            plsc.addupdate(acc.at[pl.ds(j * 16, 16)], v)

        pltpu.sync_copy(acc, out_hbm.at[pl.ds(row * OUT, OUT)])
        return carry

    lax.fori_loop(0, RPW, row_body, 0, unroll=False)


@jax.jit
def kernel(probability_distribution):
    out = _sc_kernel(probability_distribution.reshape(-1))
    return out.reshape(BATCH, OUT)


# speedup vs baseline: 3.2454x; 3.2454x over previous
import functools

import jax
import jax.numpy as jnp
from jax import lax
from jax.experimental import pallas as pl
from jax.experimental.pallas import tpu as pltpu
from jax.experimental.pallas import tpu_sc as plsc

BATCH = 1024
IN = 100000
OUT = 1024

NW = 32
GROUP_ROWS = 8
NGROUPS = BATCH // GROUP_ROWS
GPW = NGROUPS // NW
CQ = 4
CHW = CQ * OUT
NCH = 24
SC_COLS = NCH * CHW
TAIL_COLS = IN - SC_COLS

_mesh = plsc.VectorSubcoreMesh(core_axis_name="c", subcore_axis_name="s")


@functools.partial(
    pl.kernel,
    mesh=_mesh,
    out_type=jax.ShapeDtypeStruct((BATCH, OUT), jnp.float32),
    scratch_types=[
        pltpu.VMEM((GROUP_ROWS, CHW), jnp.float32),
        pltpu.VMEM((GROUP_ROWS, CHW), jnp.float32),
        pltpu.VMEM((GROUP_ROWS, OUT), jnp.float32),
        pltpu.SemaphoreType.DMA,
        pltpu.SemaphoreType.DMA,
        pltpu.SemaphoreType.DMA,
    ],
    compiler_params=pltpu.CompilerParams(use_tc_tiling_on_sc=True),
)
def _sc_kernel(x_hbm, init_hbm, out_hbm, buf0, buf1, acc, sem0, sem1, sema):
    bufs = (buf0, buf1)
    sems = (sem0, sem1)
    wid = lax.axis_index("s") * 2 + lax.axis_index("c")

    def start_chunk(g, c, b):
        pltpu.async_copy(
            x_hbm.at[pl.ds(g * GROUP_ROWS, GROUP_ROWS), pl.ds(c * CHW, CHW)],
            bufs[b], sems[b])

    def wait_chunk(b):
        pltpu.make_async_copy(
            x_hbm.at[pl.ds(0, GROUP_ROWS), pl.ds(0, CHW)],
            bufs[b], sems[b]).wait()

    def accum_chunk(b):
        buf = bufs[b]
        for s in range(GROUP_ROWS):
            def h_body(h, carry):
                v = buf[s, pl.ds(h * 16, 16)]
                for p in range(1, CQ):
                    v = v + buf[s, pl.ds(p * OUT + h * 16, 16)]
                plsc.addupdate(acc.at[s, pl.ds(h * 16, 16)], v)
                return carry

            lax.fori_loop(0, OUT // 16, h_body, 0, unroll=8)

    def group_body(gi, carry):
        g = wid * GPW + gi
        pltpu.async_copy(
            init_hbm.at[pl.ds(g * GROUP_ROWS, GROUP_ROWS), :], acc, sema)
        start_chunk(g, 0, 0)
        start_chunk(g, 1, 1)
        pltpu.make_async_copy(
            init_hbm.at[pl.ds(0, GROUP_ROWS), :], acc, sema).wait()

        def chunk_body(i, carry2):
            for b in range(2):
                c = 2 * i + b
                wait_chunk(b)
                accum_chunk(b)

                @pl.when(c + 2 < NCH)
                def _():
                    start_chunk(g, c + 2, b)

            return carry2

        lax.fori_loop(0, NCH // 2, chunk_body, 0, unroll=False)
        pltpu.sync_copy(acc, out_hbm.at[pl.ds(g * GROUP_ROWS, GROUP_ROWS), :])
        return carry

    lax.fori_loop(0, GPW, group_body, 0, unroll=False)


_TAIL_BBLK = 256


def _tail_body(x_ref, o_ref):
    x = x_ref[...]
    col = jax.lax.broadcasted_iota(jnp.int32, x.shape, 1)
    o_ref[...] = x[:, 0:OUT] + jnp.where(col < TAIL_COLS, x, 0.0)[:, OUT:2 * OUT]


def _tail_partial(x):
    return pl.pallas_call(
        _tail_body,
        grid=(BATCH // _TAIL_BBLK,),
        in_specs=[pl.BlockSpec((_TAIL_BBLK, 2 * OUT),
                               lambda i: (i, SC_COLS // (2 * OUT)))],
        out_specs=pl.BlockSpec((_TAIL_BBLK, OUT), lambda i: (i, 0)),
        out_shape=jax.ShapeDtypeStruct((BATCH, OUT), jnp.float32),
        compiler_params=pltpu.CompilerParams(
            dimension_semantics=("parallel",),
        ),
    )(x)


@jax.jit
def kernel(probability_distribution):
    init = _tail_partial(probability_distribution)
    return _sc_kernel(probability_distribution, init)
